# Initial kernel scaffold; baseline (speedup 1.0000x reference)
import jax
import jax.numpy as jnp
from jax.experimental import pallas as pl


def kernel(activation, ema_activation, pseudo_label, queue_list):
    raise NotImplementedError("write your pallas kernel here")


# speedup vs baseline: 3.0898x; 3.0898x over previous
import jax
import jax.numpy as jnp
from jax.experimental import pallas as pl

TEMPERATURE = 0.5
QUEUE_SIZE = 125
CLASS_NUM = 3
PROJ_DIM = 128
BATCH = 1024
TOTAL_Q = QUEUE_SIZE * CLASS_NUM


def _loss_kernel(act_ref, ema_ref, plabel_ref, queue_ref, out_ref):
    act = act_ref[...]
    ema = ema_ref[...]
    pl_probs = plabel_ref[...]
    queue = queue_ref[...]

    eps = 1e-12
    f = act * jax.lax.rsqrt(jnp.maximum(jnp.sum(act * act, axis=1, keepdims=True), eps * eps))
    ef = ema * jax.lax.rsqrt(jnp.maximum(jnp.sum(ema * ema, axis=1, keepdims=True), eps * eps))

    inv_t = 1.0 / TEMPERATURE
    l_pos = jnp.exp(jnp.sum(f * ef, axis=1, keepdims=True) * inv_t)

    sims = jnp.exp(
        jax.lax.dot_general(
            f, queue, (((1,), (1,)), ((), ())),
            preferred_element_type=jnp.float32,
        ) * inv_t
    )

    denom = l_pos + jnp.sum(sims, axis=1, keepdims=True)

    p0 = pl_probs[:, 0:1]
    p1 = pl_probs[:, 1:2]
    p2 = pl_probs[:, 2:3]
    lab01 = jnp.where(p0 >= p1, 0, 1)
    m01 = jnp.maximum(p0, p1)
    label = jnp.where(m01 >= p2, lab01, 2)

    col_cls = jax.lax.broadcasted_iota(jnp.int32, (BATCH, TOTAL_Q), 1) // QUEUE_SIZE
    mask = col_cls == label

    log_terms = jnp.log(sims / denom + 1e-6)
    seg = jnp.sum(jnp.where(mask, log_terms, 0.0), axis=1, keepdims=True)

    per = -(seg + jnp.log(l_pos / denom + 1e-6)) / (QUEUE_SIZE + 1)
    out_ref[...] = jnp.sum(per, axis=(0, 1), keepdims=True) / BATCH


def kernel(activation, ema_activation, pseudo_label, queue_list):
    out = pl.pallas_call(
        _loss_kernel,
        out_shape=jax.ShapeDtypeStruct((1, 1), jnp.float32),
    )(activation, ema_activation, pseudo_label, queue_list)
    return out[0, 0]
